# Initial kernel scaffold; baseline (speedup 1.0000x reference)
import jax
import jax.numpy as jnp
from jax.experimental import pallas as pl


def kernel(x, pos, batch, edge_index_3rd, params):
    raise NotImplementedError("write your pallas kernel here")


# speedup vs baseline: 2.9195x; 2.9195x over previous
import functools

import numpy as np
import jax
import jax.numpy as jnp
from jax import lax
from jax.experimental import pallas as pl
from jax.experimental.pallas import tpu as pltpu
from jax.experimental.pallas import tpu_sc as plsc

N = 50000
E = 800000
F_IN = 5
H = 64
LAT = 64
B = 64
NI = 3
CUTOFF = 10.0
EG = 68
EPS = 1e-8

EP = 802816
EBLK = 1024
EGRID = EP // EBLK
GBLK = 512
PER_TILE = EP // 32
NBLK = PER_TILE // GBLK
PER_SUB = EP // 16
NBLK2 = PER_SUB // GBLK
NSUB = 16
ROWS_PT = N // NSUB
NBLKN = 50
NBLKSZ = N // NBLKN

_f32 = jnp.float32

_mesh = plsc.VectorSubcoreMesh(core_axis_name="c", subcore_axis_name="s")
_sc_params = pltpu.CompilerParams(
    needs_layout_passes=False, use_tc_tiling_on_sc=False)


@functools.partial(
    pl.kernel,
    out_type=jax.ShapeDtypeStruct((EP, 8), _f32),
    mesh=_mesh,
    scratch_types=[
        pltpu.VMEM((GBLK,), jnp.int32),
        pltpu.VMEM((GBLK,), jnp.int32),
        pltpu.VMEM((GBLK,), jnp.int32),
        pltpu.VMEM((GBLK,), jnp.int32),
        pltpu.VMEM((GBLK, 16), _f32),
        pltpu.VMEM((GBLK, 16), _f32),
        pltpu.VMEM((GBLK, 16), _f32),
        pltpu.VMEM((GBLK, 16), _f32),
        pltpu.VMEM((GBLK, 8), _f32),
        pltpu.SemaphoreType.DMA,
    ],
    compiler_params=_sc_params,
)
def _geo(idx_hbm, pos_hbm, out_hbm, i0, i1, i2, i3, p0, p1, p2, p3, ob, sem):
    c = lax.axis_index("c")
    s = lax.axis_index("s")
    base0 = (s * 2 + c) * PER_TILE

    def blk(b, carry):
        base = base0 + b * GBLK
        pltpu.sync_copy(idx_hbm.at[0, pl.ds(base, GBLK)], i0)
        pltpu.sync_copy(idx_hbm.at[1, pl.ds(base, GBLK)], i1)
        pltpu.sync_copy(idx_hbm.at[2, pl.ds(base, GBLK)], i2)
        pltpu.sync_copy(idx_hbm.at[3, pl.ds(base, GBLK)], i3)
        cps = [
            pltpu.async_copy(pos_hbm.at[i0], p0, sem),
            pltpu.async_copy(pos_hbm.at[i1], p1, sem),
            pltpu.async_copy(pos_hbm.at[i2], p2, sem),
            pltpu.async_copy(pos_hbm.at[i3], p3, sem),
        ]
        for cp in cps:
            cp.wait()
        for g in range(GBLK // 16):
            rid = lax.iota(jnp.int32, 16) + g * 16

            def ld(ref, comp):
                return plsc.load_gather(
                    ref, [rid, jnp.full((16,), comp, jnp.int32)])

            pix, piy, piz = ld(p0, 0), ld(p0, 1), ld(p0, 2)
            pjx, pjy, pjz = ld(p1, 0), ld(p1, 1), ld(p1, 2)
            pkx, pky, pkz = ld(p2, 0), ld(p2, 1), ld(p2, 2)
            ptx, pty, ptz = ld(p3, 0), ld(p3, 1), ld(p3, 2)
            b1x, b1y, b1z = pjx - pix, pjy - piy, pjz - piz
            b2x, b2y, b2z = pkx - pjx, pky - pjy, pkz - pjz
            b3x, b3y, b3z = ptx - pkx, pty - pky, ptz - pkz
            A = b1x * b1x + b1y * b1y + b1z * b1z
            P = b1x * b2x + b1y * b2y + b1z * b2z
            n1x = b1y * b2z - b1z * b2y
            n1y = b1z * b2x - b1x * b2z
            n1z = b1x * b2y - b1y * b2x
            Q = n1x * n1x + n1y * n1y + n1z * n1z
            R = b2x * b2x + b2y * b2y + b2z * b2z
            n2x = b2y * b3z - b2z * b3y
            n2y = b2z * b3x - b2x * b3z
            n2z = b2x * b3y - b2y * b3x
            C = n1x * n2x + n1y * n2y + n1z * n2z
            cx = n1y * n2z - n1z * n2y
            cy = n1z * n2x - n1x * n2z
            cz = n1x * n2y - n1y * n2x
            S = cx * b2x + cy * b2y + cz * b2z

            def st(v, comp):
                plsc.store_scatter(
                    ob, [rid, jnp.full((16,), comp, jnp.int32)], v)

            st(A, 0)
            st(P, 1)
            st(Q, 2)
            st(R, 3)
            st(C, 4)
            st(S, 5)
        pltpu.sync_copy(ob, out_hbm.at[pl.ds(base, GBLK)])
        return carry

    lax.fori_loop(0, NBLK, blk, 0)


@functools.partial(
    pl.kernel,
    out_type=jax.ShapeDtypeStruct((2, N, 16), _f32),
    mesh=_mesh,
    scratch_types=[
        pltpu.VMEM((GBLK,), jnp.int32),
        pltpu.VMEM((GBLK,), jnp.int32),
        pltpu.VMEM((GBLK, 16), _f32),
        pltpu.VMEM((GBLK, 16), _f32),
        pltpu.VMEM_SHARED((N, 16), _f32),
        pltpu.SemaphoreType.DMA,
    ],
    compiler_params=_sc_params,
)
def _msg(ha_hbm, hb_hbm, w2_hbm, idx_hbm, zer_hbm, out_hbm,
         iv, jv, hj, wv, agg, sem):
    c = lax.axis_index("c")
    s = lax.axis_index("s")
    r0 = s * ROWS_PT
    pltpu.sync_copy(zer_hbm.at[pl.ds(r0, ROWS_PT)], agg.at[pl.ds(r0, ROWS_PT)])
    plsc.subcore_barrier()
    base0 = s * PER_SUB

    def blk(b, carry):
        base = base0 + b * GBLK
        pltpu.sync_copy(idx_hbm.at[0, pl.ds(base, GBLK)], iv)
        pltpu.sync_copy(idx_hbm.at[1, pl.ds(base, GBLK)], jv)

        @pl.when(c == 0)
        def _():
            pltpu.async_copy(ha_hbm.at[jv], hj, sem).wait()

        @pl.when(c == 1)
        def _():
            pltpu.async_copy(hb_hbm.at[jv], hj, sem).wait()

        pltpu.sync_copy(w2_hbm.at[c, pl.ds(base, GBLK)], wv)

        def mul(r, carry2):
            r8 = r * 8
            for u in range(8):
                hj[r8 + u, 0:16] = hj[r8 + u, 0:16] * wv[r8 + u, 0:16]
            return carry2

        lax.fori_loop(0, GBLK // 8, mul, 0)
        pltpu.sync_copy(hj, agg.at[iv], add=True)
        return carry

    lax.fori_loop(0, NBLK2, blk, 0)
    plsc.subcore_barrier()
    pltpu.sync_copy(agg.at[pl.ds(r0, ROWS_PT)], out_hbm.at[c, pl.ds(r0, ROWS_PT)])


_QSPEC = pl.BlockSpec((NBLKSZ, 16), lambda g: (g, 0))
_QSHAPE = jax.ShapeDtypeStruct((N, 16), _f32)


def _hinit_body(x_ref, w_ref, b_ref, q0, q1, q2, q3):
    h = jnp.dot(x_ref[...], w_ref[...], preferred_element_type=_f32) + b_ref[...]
    q0[...] = h[:, 0:16]
    q1[...] = h[:, 16:32]
    q2[...] = h[:, 32:48]
    q3[...] = h[:, 48:64]


def _hinit(x, w_emb, b_emb):
    return pl.pallas_call(
        _hinit_body,
        grid=(NBLKN,),
        in_specs=[
            pl.BlockSpec((NBLKSZ, F_IN), lambda g: (g, 0)),
            pl.BlockSpec((F_IN, H), lambda g: (0, 0)),
            pl.BlockSpec((1, H), lambda g: (0, 0)),
        ],
        out_specs=[_QSPEC, _QSPEC, _QSPEC, _QSPEC],
        out_shape=[_QSHAPE, _QSHAPE, _QSHAPE, _QSHAPE],
    )(x, w_emb, b_emb)


_STEP_D = np.float32(CUTOFF / 49.0)
_STEP_A = np.float32(np.pi / 5.0)
_STEP_T = np.float32(2.0 * np.pi / 11.0)
_CO_D = np.float32(-0.5) / _STEP_D ** 2
_CO_A = np.float32(-0.5) / _STEP_A ** 2
_CO_T = np.float32(-0.5) / _STEP_T ** 2


def _smear(v, num, start, step, coeff):
    off = lax.broadcasted_iota(jnp.int32, (1, num), 1).astype(_f32) * step + start
    return jnp.exp(coeff * (v - off) ** 2)


def _edgew_body(geo_ref, we1_ref, be1_ref, we2_ref, be2_ref, *w_refs):
    g = pl.program_id(0)
    geo = geo_ref[...]
    A = geo[:, 0:1]
    P = geo[:, 1:2]
    Q = geo[:, 2:3]
    R = geo[:, 3:4]
    C = geo[:, 4:5]
    S = geo[:, 5:6]
    d = jnp.sqrt(A + EPS)
    ang = jnp.arctan2(jnp.sqrt(Q + EPS), P)
    tor = jnp.arctan2(S * lax.rsqrt(R + EPS), C)
    e = jnp.concatenate([
        _smear(d, 50, np.float32(0.0), _STEP_D, _CO_D),
        _smear(ang, 6, np.float32(0.0), _STEP_A, _CO_A),
        _smear(tor, 12, np.float32(-np.pi), _STEP_T, _CO_T),
    ], axis=1)
    valid = (g * EBLK + lax.broadcasted_iota(jnp.int32, (EBLK, 1), 0)) < E
    for l in range(NI):
        hid = jax.nn.relu(
            jnp.dot(e, we1_ref[l], preferred_element_type=_f32) + be1_ref[l])
        w = jnp.dot(hid, we2_ref[l], preferred_element_type=_f32) + be2_ref[l]
        w = jnp.where(valid, w, 0.0)
        w_refs[2 * l][0] = w[:, 0:16]
        w_refs[2 * l][1] = w[:, 32:48]
        w_refs[2 * l + 1][0] = w[:, 16:32]
        w_refs[2 * l + 1][1] = w[:, 48:64]


def _edgew(geo, we1s, be1s, we2s, be2s):
    wspec = pl.BlockSpec((2, EBLK, 16), lambda g: (0, g, 0))
    wshape = jax.ShapeDtypeStruct((2, EP, 16), _f32)
    return pl.pallas_call(
        _edgew_body,
        grid=(EGRID,),
        in_specs=[
            pl.BlockSpec((EBLK, 8), lambda g: (g, 0)),
            pl.BlockSpec((NI, EG, H), lambda g: (0, 0, 0)),
            pl.BlockSpec((NI, 1, H), lambda g: (0, 0, 0)),
            pl.BlockSpec((NI, H, H), lambda g: (0, 0, 0)),
            pl.BlockSpec((NI, 1, H), lambda g: (0, 0, 0)),
        ],
        out_specs=[wspec] * (2 * NI),
        out_shape=[wshape] * (2 * NI),
    )(geo, we1s, be1s, we2s, be2s)


_AGGSPEC = pl.BlockSpec((2, NBLKSZ, 16), lambda g: (0, g, 0))


def _upd_body(q0, q1, q2, q3, aggA, aggB, wu_ref, bu_ref, o0, o1, o2, o3):
    h = jnp.concatenate([q0[...], q1[...], q2[...], q3[...]], axis=1)
    agg = jnp.concatenate([aggA[0], aggB[0], aggA[1], aggB[1]], axis=1)
    hn = h + jax.nn.relu(
        jnp.dot(agg, wu_ref[...], preferred_element_type=_f32) + bu_ref[...])
    o0[...] = hn[:, 0:16]
    o1[...] = hn[:, 16:32]
    o2[...] = hn[:, 32:48]
    o3[...] = hn[:, 48:64]


def _upd(hq, aggA, aggB, wu, bu):
    return pl.pallas_call(
        _upd_body,
        grid=(NBLKN,),
        in_specs=[
            _QSPEC, _QSPEC, _QSPEC, _QSPEC,
            _AGGSPEC, _AGGSPEC,
            pl.BlockSpec((H, H), lambda g: (0, 0)),
            pl.BlockSpec((1, H), lambda g: (0, 0)),
        ],
        out_specs=[_QSPEC, _QSPEC, _QSPEC, _QSPEC],
        out_shape=[_QSHAPE, _QSHAPE, _QSHAPE, _QSHAPE],
    )(*hq, aggA, aggB, wu, bu)


def _readout_body(q0, q1, q2, q3, bat_ref, l1w_ref, l1b_ref,
                  wc1_ref, bc1_ref, wc2_ref, bc2_ref, out_ref, acc_ref):
    g = pl.program_id(0)
    h = jnp.concatenate([q0[...], q1[...], q2[...], q3[...]], axis=1)
    zn = jnp.dot(h, l1w_ref[...], preferred_element_type=_f32) + l1b_ref[...]
    onehot = (bat_ref[...] ==
              lax.broadcasted_iota(jnp.int32, (NBLKSZ, B), 1)).astype(_f32)
    zpart = lax.dot_general(onehot, zn, (((0,), (0,)), ((), ())),
                            preferred_element_type=_f32)

    @pl.when(g == 0)
    def _():
        acc_ref[...] = jnp.zeros_like(acc_ref)

    acc_ref[...] += zpart
    hid = jax.nn.relu(
        jnp.dot(acc_ref[...], wc1_ref[...], preferred_element_type=_f32)
        + bc1_ref[...])
    out_ref[...] = jnp.dot(hid, wc2_ref[...],
                           preferred_element_type=_f32) + bc2_ref[...]


def _readout(hq, bat, l1w, l1b, wc1, bc1, wc2, bc2):
    return pl.pallas_call(
        _readout_body,
        grid=(NBLKN,),
        in_specs=[
            _QSPEC, _QSPEC, _QSPEC, _QSPEC,
            pl.BlockSpec((NBLKSZ, 1), lambda g: (g, 0)),
            pl.BlockSpec((H, LAT), lambda g: (0, 0)),
            pl.BlockSpec((1, LAT), lambda g: (0, 0)),
            pl.BlockSpec((LAT, H // 2), lambda g: (0, 0)),
            pl.BlockSpec((1, H // 2), lambda g: (0, 0)),
            pl.BlockSpec((H // 2, 2), lambda g: (0, 0)),
            pl.BlockSpec((1, 2), lambda g: (0, 0)),
        ],
        out_specs=pl.BlockSpec((B, 2), lambda g: (0, 0)),
        out_shape=jax.ShapeDtypeStruct((B, 2), _f32),
        scratch_shapes=[pltpu.VMEM((B, B), _f32)],
    )(*hq, bat, l1w, l1b, wc1, bc1, wc2, bc2)


def kernel(x, pos, batch, edge_index_3rd, params):
    idx32 = jnp.pad(edge_index_3rd.astype(jnp.int32), ((0, 0), (0, EP - E)))
    pos_pad = jnp.zeros((N, 16), _f32).at[:, :3].set(pos.astype(_f32))
    bat32 = batch.astype(jnp.int32).reshape(N, 1)
    zer16 = jnp.zeros((N, 16), _f32)

    geo = _geo(idx32, pos_pad)
    hq = list(_hinit(x, params["W_emb"], params["b_emb"].reshape(1, H)))

    we1s = jnp.stack([params[f"We1_{l}"] for l in range(NI)])
    be1s = jnp.stack([params[f"be1_{l}"].reshape(1, H) for l in range(NI)])
    we2s = jnp.stack([params[f"We2_{l}"] for l in range(NI)])
    be2s = jnp.stack([params[f"be2_{l}"].reshape(1, H) for l in range(NI)])
    ws = _edgew(geo, we1s, be1s, we2s, be2s)

    for l in range(NI):
        aggA = _msg(hq[0], hq[2], ws[2 * l], idx32, zer16)
        aggB = _msg(hq[1], hq[3], ws[2 * l + 1], idx32, zer16)
        hq = list(_upd(hq, aggA, aggB, params[f"Wu_{l}"],
                       params[f"bu_{l}"].reshape(1, H)))

    return _readout(hq, bat32,
                    params["lin1_W"], params["lin1_b"].reshape(1, LAT),
                    params["Wc1"], params["bc1"].reshape(1, H // 2),
                    params["Wc2"], params["bc2"].reshape(1, 2))
